# baseline (speedup 1.0000x reference)
import functools

import jax
import jax.numpy as jnp
from jax import lax
from jax.experimental import pallas as pl
from jax.experimental.pallas import tpu as pltpu
from jax.experimental.pallas import tpu_sc as plsc

N = 10000
E = 320000
D = 128
TILES = 16
CORES = 2
CHUNK = 128
NCHUNK = 80
NC0 = 40
NC1 = 120
NT = (NC0 + NC1) * TILES
EPT = NCHUNK * CHUNK
E_PAD = EPT * TILES * CORES
N_ACC = 10240
PAD_DST = 10008
ROWS_T = 624
ROWS_LAST = N - 15 * ROWS_T
ACC_T = N_ACC // TILES

_MESH = plsc.VectorSubcoreMesh(core_axis_name="c", subcore_axis_name="s")


@functools.partial(
    pl.kernel,
    out_type=jax.ShapeDtypeStruct((CORES, N_ACC), jnp.float32),
    mesh=_MESH,
    scratch_types=[
        pltpu.VMEM((NCHUNK, CHUNK), jnp.int32),
        pltpu.VMEM((CHUNK,), jnp.float32),
        pltpu.VMEM((ACC_T,), jnp.float32),
        pltpu.VMEM_SHARED((N_ACC,), jnp.float32),
    ],
)
def _sc_degree(dst_hbm, deg_hbm, dst_v, ones_v, zer_v, deg_sh):
    c = lax.axis_index("c")
    s = lax.axis_index("s")
    for i in range(CHUNK // 16):
        ones_v[pl.ds(i * 16, 16)] = jnp.ones((16,), jnp.float32)
    def _z(i, _):
        zer_v[pl.ds(i * 16, 16)] = jnp.zeros((16,), jnp.float32)
        return 0
    lax.fori_loop(0, ACC_T // 16, _z, 0)
    pltpu.sync_copy(zer_v, deg_sh.at[pl.ds(s * ACC_T, ACC_T)])
    pltpu.sync_copy(
        dst_hbm.at[pl.ds((c * TILES + s) * NCHUNK, NCHUNK), :], dst_v)
    plsc.subcore_barrier()

    def _body(j, _):
        pltpu.sync_copy(ones_v, deg_sh.at[dst_v.at[j]], add=True)
        return 0
    lax.fori_loop(0, NCHUNK, _body, 0)
    plsc.subcore_barrier()
    pltpu.sync_copy(deg_sh.at[pl.ds(s * ACC_T, ACC_T)],
                    deg_hbm.at[c, pl.ds(s * ACC_T, ACC_T)])


@functools.partial(
    pl.kernel,
    out_type=jax.ShapeDtypeStruct((CORES, N, D), jnp.float32),
    mesh=_MESH,
    scratch_types=[
        pltpu.VMEM((CHUNK,), jnp.int32),
        pltpu.VMEM((CHUNK,), jnp.int32),
        pltpu.VMEM((1, CHUNK), jnp.int32),
        pltpu.VMEM((1, CHUNK), jnp.int32),
        pltpu.VMEM((CHUNK, D), jnp.float32),
        pltpu.VMEM((CHUNK, D), jnp.float32),
        pltpu.VMEM((8, D), jnp.float32),
        pltpu.VMEM_SHARED((N_ACC, D), jnp.float32),
        pltpu.SemaphoreType.DMA,
        pltpu.SemaphoreType.DMA,
    ],
)
def _sc_message(hp_hbm, src_hbm, dst_hbm, out_hbm,
                sidx0_v, sidx1_v, didx0_v, didx1_v, rows0_v, rows1_v,
                zer_v, acc_sh, sem_g, sem_i):
    c = lax.axis_index("c")
    s = lax.axis_index("s")
    def _z(i, _):
        zer_v[i // 8, pl.ds((i % 8) * 16, 16)] = jnp.zeros((16,), jnp.float32)
        return 0
    lax.fori_loop(0, 8 * D // 16, _z, 0)

    def _zacc(j, _):
        pltpu.sync_copy(zer_v, acc_sh.at[pl.ds(s * ACC_T + j * 8, 8), :])
        return 0
    lax.fori_loop(0, ACC_T // 8, _zacc, 0)
    plsc.subcore_barrier()

    def _fetch_idx(t, ibuf, dbuf):
        pltpu.async_copy(src_hbm.at[pl.ds(t * CHUNK, CHUNK)], ibuf, sem_i)
        pltpu.async_copy(dst_hbm.at[t], dbuf, sem_i)

    def _iwait(ibuf, dbuf):
        pltpu.make_async_copy(src_hbm.at[pl.ds(0, CHUNK)], ibuf, sem_i).wait()
        pltpu.make_async_copy(dst_hbm.at[0], dbuf, sem_i).wait()

    def _gather(ibuf, buf):
        pltpu.async_copy(hp_hbm.at[ibuf], buf, sem_g)

    def _gwait(ibuf, buf):
        pltpu.make_async_copy(hp_hbm.at[ibuf], buf, sem_g).wait()

    def _run(nck, t0):
        _fetch_idx(t0, sidx0_v, didx0_v)
        _iwait(sidx0_v, didx0_v)
        _gather(sidx0_v, rows0_v)
        if nck > 1:
            _fetch_idx(t0 + 1, sidx1_v, didx1_v)

        def _body(g, _):
            def _step(cur_i, cur_d, cur_r, nxt_i, nxt_d, nxt_r):
                _gwait(cur_i, cur_r)

                @pl.when(g + 1 < nck)
                def _launch_next():
                    _iwait(nxt_i, nxt_d)
                    _gather(nxt_i, nxt_r)

                @pl.when(g + 2 < nck)
                def _prefetch_idx():
                    _fetch_idx(t0 + g + 2, cur_i, cur_d)
                pltpu.sync_copy(cur_r, acc_sh.at[cur_d.at[0]], add=True)

            @pl.when(g % 2 == 0)
            def _even():
                _step(sidx0_v, didx0_v, rows0_v, sidx1_v, didx1_v, rows1_v)

            @pl.when(g % 2 == 1)
            def _odd():
                _step(sidx1_v, didx1_v, rows1_v, sidx0_v, didx0_v, rows0_v)
            return 0
        lax.fori_loop(0, nck, _body, 0)

    @pl.when(c == 0)
    def _core0():
        _run(NC0, s * NC0)

    @pl.when(c == 1)
    def _core1():
        _run(NC1, TILES * NC0 + s * NC1)
    plsc.subcore_barrier()

    @pl.when(s < 15)
    def _drain():
        pltpu.sync_copy(acc_sh.at[pl.ds(s * ROWS_T, ROWS_T), :],
                        out_hbm.at[c, pl.ds(s * ROWS_T, ROWS_T), :])

    @pl.when(s == 15)
    def _drain_last():
        pltpu.sync_copy(acc_sh.at[pl.ds(15 * ROWS_T, ROWS_LAST), :],
                        out_hbm.at[c, pl.ds(15 * ROWS_T, ROWS_LAST), :])


_BM = 1000


def _tc_first(x, W1, deg):
    def body(x_ref, w_ref, d_ref, o_ref):
        dinv = lax.rsqrt(d_ref[0] + d_ref[1] + 1.0)
        o_ref[...] = jnp.dot(x_ref[...], w_ref[...],
                             preferred_element_type=jnp.float32) * dinv
    return pl.pallas_call(
        body,
        grid=(N // _BM,),
        in_specs=[
            pl.BlockSpec((_BM, D), lambda i: (i, 0)),
            pl.BlockSpec((D, D), lambda i: (0, 0)),
            pl.BlockSpec((CORES, _BM, 1), lambda i: (0, i, 0)),
        ],
        out_specs=pl.BlockSpec((_BM, D), lambda i: (i, 0)),
        out_shape=jax.ShapeDtypeStruct((N, D), jnp.float32),
    )(x, W1, deg)


def _tc_mid(acc, hp, deg, b, W2):
    def body(a_ref, h_ref, d_ref, b_ref, w_ref, o_ref):
        dinv = lax.rsqrt(d_ref[0] + d_ref[1] + 1.0)
        tot = a_ref[0] + a_ref[1] + h_ref[...]
        x2 = jnp.maximum(tot * dinv + b_ref[...], 0.0)
        o_ref[...] = jnp.dot(x2, w_ref[...],
                             preferred_element_type=jnp.float32) * dinv
    return pl.pallas_call(
        body,
        grid=(N // _BM,),
        in_specs=[
            pl.BlockSpec((CORES, _BM, D), lambda i: (0, i, 0)),
            pl.BlockSpec((_BM, D), lambda i: (i, 0)),
            pl.BlockSpec((CORES, _BM, 1), lambda i: (0, i, 0)),
            pl.BlockSpec((1, D), lambda i: (0, 0)),
            pl.BlockSpec((D, D), lambda i: (0, 0)),
        ],
        out_specs=pl.BlockSpec((_BM, D), lambda i: (i, 0)),
        out_shape=jax.ShapeDtypeStruct((N, D), jnp.float32),
    )(acc, hp, deg, b, W2)


def _tc_last(acc, hp, deg, b, Wfm, Wfx, bfc):
    def body(a_ref, h_ref, d_ref, b_ref, wm_ref, wx_ref, bf_ref, o_ref,
             sum_ref, max_ref):
        i = pl.program_id(0)
        dinv = lax.rsqrt(d_ref[0] + d_ref[1] + 1.0)
        tot = a_ref[0] + a_ref[1] + h_ref[...]
        x3 = jnp.maximum(tot * dinv + b_ref[...], 0.0)
        bsum = jnp.sum(x3, axis=0, keepdims=True)
        bmax = jnp.max(x3, axis=0, keepdims=True)

        @pl.when(i == 0)
        def _init():
            sum_ref[...] = bsum
            max_ref[...] = bmax

        @pl.when(i > 0)
        def _accum():
            sum_ref[...] += bsum
            max_ref[...] = jnp.maximum(max_ref[...], bmax)

        @pl.when(i == N // _BM - 1)
        def _final():
            mean = sum_ref[...] * (1.0 / N)
            o_ref[...] = (jnp.dot(mean, wm_ref[...],
                                  preferred_element_type=jnp.float32)
                          + jnp.dot(max_ref[...], wx_ref[...],
                                    preferred_element_type=jnp.float32)
                          + bf_ref[...])
    return pl.pallas_call(
        body,
        grid=(N // _BM,),
        in_specs=[
            pl.BlockSpec((CORES, _BM, D), lambda i: (0, i, 0)),
            pl.BlockSpec((_BM, D), lambda i: (i, 0)),
            pl.BlockSpec((CORES, _BM, 1), lambda i: (0, i, 0)),
            pl.BlockSpec((1, D), lambda i: (0, 0)),
            pl.BlockSpec((D, D), lambda i: (0, 0)),
            pl.BlockSpec((D, D), lambda i: (0, 0)),
            pl.BlockSpec((1, D), lambda i: (0, 0)),
        ],
        out_specs=pl.BlockSpec((1, D), lambda i: (0, 0)),
        out_shape=jax.ShapeDtypeStruct((1, D), jnp.float32),
        scratch_shapes=[
            pltpu.VMEM((1, D), jnp.float32),
            pltpu.VMEM((1, D), jnp.float32),
        ],
    )(acc, hp, deg, b, Wfm, Wfx, bfc)


def kernel(node_features, edge_index, W1, b1, W2, b2, W_fc, b_fc):
    ei = edge_index.astype(jnp.int32)
    npad = E_PAD - E
    src = jnp.concatenate([ei[0], jnp.zeros((npad,), jnp.int32)])
    dst = jnp.concatenate([ei[1], jnp.full((npad,), PAD_DST, jnp.int32)])
    dst2d = dst.reshape(E_PAD // CHUNK, CHUNK)
    dst3d = dst.reshape(NT, 1, CHUNK)

    deg = _sc_degree(dst2d)[:, :N].reshape(CORES, N, 1)
    hp1 = _tc_first(node_features, W1, deg)
    acc1 = _sc_message(hp1, src, dst3d)
    hp2 = _tc_mid(acc1, hp1, deg, b1.reshape(1, D), W2)
    acc2 = _sc_message(hp2, src, dst3d)
    return _tc_last(acc2, hp2, deg, b2.reshape(1, D),
                    W_fc[:D], W_fc[D:], b_fc.reshape(1, D))


# speedup vs baseline: 1.1706x; 1.1706x over previous
import functools

import jax
import jax.numpy as jnp
from jax import lax
from jax.experimental import pallas as pl
from jax.experimental.pallas import tpu as pltpu
from jax.experimental.pallas import tpu_sc as plsc

N = 10000
E = 320000
D = 128
TILES = 16
CORES = 2
CHUNK = 128
NCHUNK = 80
NC0 = 120
NC1 = 40
NT = (NC0 + NC1) * TILES
EPT = NCHUNK * CHUNK
E_PAD = EPT * TILES * CORES
N_ACC = 10240
PAD_DST = 10008
ROWS_T = 624
ROWS_LAST = N - 15 * ROWS_T
ACC_T = N_ACC // TILES

_MESH = plsc.VectorSubcoreMesh(core_axis_name="c", subcore_axis_name="s")


@functools.partial(
    pl.kernel,
    out_type=jax.ShapeDtypeStruct((CORES, N_ACC), jnp.float32),
    mesh=_MESH,
    scratch_types=[
        pltpu.VMEM((NCHUNK, CHUNK), jnp.int32),
        pltpu.VMEM((CHUNK,), jnp.float32),
        pltpu.VMEM((ACC_T,), jnp.float32),
        pltpu.VMEM_SHARED((N_ACC,), jnp.float32),
    ],
)
def _sc_degree(dst_hbm, deg_hbm, dst_v, ones_v, zer_v, deg_sh):
    c = lax.axis_index("c")
    s = lax.axis_index("s")
    for i in range(CHUNK // 16):
        ones_v[pl.ds(i * 16, 16)] = jnp.ones((16,), jnp.float32)
    def _z(i, _):
        zer_v[pl.ds(i * 16, 16)] = jnp.zeros((16,), jnp.float32)
        return 0
    lax.fori_loop(0, ACC_T // 16, _z, 0)
    pltpu.sync_copy(zer_v, deg_sh.at[pl.ds(s * ACC_T, ACC_T)])
    pltpu.sync_copy(
        dst_hbm.at[pl.ds((c * TILES + s) * NCHUNK, NCHUNK), :], dst_v)
    plsc.subcore_barrier()

    def _body(j, _):
        pltpu.sync_copy(ones_v, deg_sh.at[dst_v.at[j]], add=True)
        return 0
    lax.fori_loop(0, NCHUNK, _body, 0)
    plsc.subcore_barrier()
    pltpu.sync_copy(deg_sh.at[pl.ds(s * ACC_T, ACC_T)],
                    deg_hbm.at[c, pl.ds(s * ACC_T, ACC_T)])


@functools.partial(
    pl.kernel,
    out_type=jax.ShapeDtypeStruct((CORES, N, D), jnp.float32),
    mesh=_MESH,
    scratch_types=[
        pltpu.VMEM((CHUNK,), jnp.int32),
        pltpu.VMEM((CHUNK,), jnp.int32),
        pltpu.VMEM((1, CHUNK), jnp.int32),
        pltpu.VMEM((1, CHUNK), jnp.int32),
        pltpu.VMEM((CHUNK, D), jnp.float32),
        pltpu.VMEM((CHUNK, D), jnp.float32),
        pltpu.VMEM((8, D), jnp.float32),
        pltpu.VMEM_SHARED((N_ACC, D), jnp.float32),
        pltpu.SemaphoreType.DMA,
        pltpu.SemaphoreType.DMA,
    ],
)
def _sc_message(hp_hbm, src_hbm, dst_hbm, out_hbm,
                sidx0_v, sidx1_v, didx0_v, didx1_v, rows0_v, rows1_v,
                zer_v, acc_sh, sem_g, sem_i):
    c = lax.axis_index("c")
    s = lax.axis_index("s")
    def _z(i, _):
        zer_v[i // 8, pl.ds((i % 8) * 16, 16)] = jnp.zeros((16,), jnp.float32)
        return 0
    lax.fori_loop(0, 8 * D // 16, _z, 0)

    def _zacc(j, _):
        pltpu.sync_copy(zer_v, acc_sh.at[pl.ds(s * ACC_T + j * 8, 8), :])
        return 0
    lax.fori_loop(0, ACC_T // 8, _zacc, 0)
    plsc.subcore_barrier()

    def _fetch_idx(t, ibuf, dbuf):
        pltpu.async_copy(src_hbm.at[pl.ds(t * CHUNK, CHUNK)], ibuf, sem_i)
        pltpu.async_copy(dst_hbm.at[t], dbuf, sem_i)

    def _iwait(ibuf, dbuf):
        pltpu.make_async_copy(src_hbm.at[pl.ds(0, CHUNK)], ibuf, sem_i).wait()
        pltpu.make_async_copy(dst_hbm.at[0], dbuf, sem_i).wait()

    def _gather(ibuf, buf):
        pltpu.async_copy(hp_hbm.at[ibuf], buf, sem_g)

    def _gwait(ibuf, buf):
        pltpu.make_async_copy(hp_hbm.at[ibuf], buf, sem_g).wait()

    def _run(nck, t0):
        _fetch_idx(t0, sidx0_v, didx0_v)
        _iwait(sidx0_v, didx0_v)
        _gather(sidx0_v, rows0_v)
        if nck > 1:
            _fetch_idx(t0 + 1, sidx1_v, didx1_v)

        def _body(g, _):
            def _step(cur_i, cur_d, cur_r, nxt_i, nxt_d, nxt_r):
                _gwait(cur_i, cur_r)

                @pl.when(g + 1 < nck)
                def _launch_next():
                    _iwait(nxt_i, nxt_d)
                    _gather(nxt_i, nxt_r)

                @pl.when(g + 2 < nck)
                def _prefetch_idx():
                    _fetch_idx(t0 + g + 2, cur_i, cur_d)
                pltpu.sync_copy(cur_r, acc_sh.at[cur_d.at[0]], add=True)

            @pl.when(g % 2 == 0)
            def _even():
                _step(sidx0_v, didx0_v, rows0_v, sidx1_v, didx1_v, rows1_v)

            @pl.when(g % 2 == 1)
            def _odd():
                _step(sidx1_v, didx1_v, rows1_v, sidx0_v, didx0_v, rows0_v)
            return 0
        lax.fori_loop(0, nck, _body, 0)

    @pl.when(c == 0)
    def _core0():
        _run(NC0, s * NC0)

    @pl.when(c == 1)
    def _core1():
        _run(NC1, TILES * NC0 + s * NC1)
    plsc.subcore_barrier()

    @pl.when(s < 15)
    def _drain():
        pltpu.sync_copy(acc_sh.at[pl.ds(s * ROWS_T, ROWS_T), :],
                        out_hbm.at[c, pl.ds(s * ROWS_T, ROWS_T), :])

    @pl.when(s == 15)
    def _drain_last():
        pltpu.sync_copy(acc_sh.at[pl.ds(15 * ROWS_T, ROWS_LAST), :],
                        out_hbm.at[c, pl.ds(15 * ROWS_T, ROWS_LAST), :])


_BM = 1000


def _tc_first(x, W1, deg):
    def body(x_ref, w_ref, d_ref, o_ref):
        dinv = lax.rsqrt(d_ref[0] + d_ref[1] + 1.0)
        o_ref[...] = jnp.dot(x_ref[...], w_ref[...],
                             preferred_element_type=jnp.float32) * dinv
    return pl.pallas_call(
        body,
        grid=(N // _BM,),
        in_specs=[
            pl.BlockSpec((_BM, D), lambda i: (i, 0)),
            pl.BlockSpec((D, D), lambda i: (0, 0)),
            pl.BlockSpec((CORES, _BM, 1), lambda i: (0, i, 0)),
        ],
        out_specs=pl.BlockSpec((_BM, D), lambda i: (i, 0)),
        out_shape=jax.ShapeDtypeStruct((N, D), jnp.float32),
    )(x, W1, deg)


def _tc_mid(acc, hp, deg, b, W2):
    def body(a_ref, h_ref, d_ref, b_ref, w_ref, o_ref):
        dinv = lax.rsqrt(d_ref[0] + d_ref[1] + 1.0)
        tot = a_ref[0] + a_ref[1] + h_ref[...]
        x2 = jnp.maximum(tot * dinv + b_ref[...], 0.0)
        o_ref[...] = jnp.dot(x2, w_ref[...],
                             preferred_element_type=jnp.float32) * dinv
    return pl.pallas_call(
        body,
        grid=(N // _BM,),
        in_specs=[
            pl.BlockSpec((CORES, _BM, D), lambda i: (0, i, 0)),
            pl.BlockSpec((_BM, D), lambda i: (i, 0)),
            pl.BlockSpec((CORES, _BM, 1), lambda i: (0, i, 0)),
            pl.BlockSpec((1, D), lambda i: (0, 0)),
            pl.BlockSpec((D, D), lambda i: (0, 0)),
        ],
        out_specs=pl.BlockSpec((_BM, D), lambda i: (i, 0)),
        out_shape=jax.ShapeDtypeStruct((N, D), jnp.float32),
    )(acc, hp, deg, b, W2)


def _tc_last(acc, hp, deg, b, Wfm, Wfx, bfc):
    def body(a_ref, h_ref, d_ref, b_ref, wm_ref, wx_ref, bf_ref, o_ref,
             sum_ref, max_ref):
        i = pl.program_id(0)
        dinv = lax.rsqrt(d_ref[0] + d_ref[1] + 1.0)
        tot = a_ref[0] + a_ref[1] + h_ref[...]
        x3 = jnp.maximum(tot * dinv + b_ref[...], 0.0)
        bsum = jnp.sum(x3, axis=0, keepdims=True)
        bmax = jnp.max(x3, axis=0, keepdims=True)

        @pl.when(i == 0)
        def _init():
            sum_ref[...] = bsum
            max_ref[...] = bmax

        @pl.when(i > 0)
        def _accum():
            sum_ref[...] += bsum
            max_ref[...] = jnp.maximum(max_ref[...], bmax)

        @pl.when(i == N // _BM - 1)
        def _final():
            mean = sum_ref[...] * (1.0 / N)
            o_ref[...] = (jnp.dot(mean, wm_ref[...],
                                  preferred_element_type=jnp.float32)
                          + jnp.dot(max_ref[...], wx_ref[...],
                                    preferred_element_type=jnp.float32)
                          + bf_ref[...])
    return pl.pallas_call(
        body,
        grid=(N // _BM,),
        in_specs=[
            pl.BlockSpec((CORES, _BM, D), lambda i: (0, i, 0)),
            pl.BlockSpec((_BM, D), lambda i: (i, 0)),
            pl.BlockSpec((CORES, _BM, 1), lambda i: (0, i, 0)),
            pl.BlockSpec((1, D), lambda i: (0, 0)),
            pl.BlockSpec((D, D), lambda i: (0, 0)),
            pl.BlockSpec((D, D), lambda i: (0, 0)),
            pl.BlockSpec((1, D), lambda i: (0, 0)),
        ],
        out_specs=pl.BlockSpec((1, D), lambda i: (0, 0)),
        out_shape=jax.ShapeDtypeStruct((1, D), jnp.float32),
        scratch_shapes=[
            pltpu.VMEM((1, D), jnp.float32),
            pltpu.VMEM((1, D), jnp.float32),
        ],
    )(acc, hp, deg, b, Wfm, Wfx, bfc)


def kernel(node_features, edge_index, W1, b1, W2, b2, W_fc, b_fc):
    ei = edge_index.astype(jnp.int32)
    npad = E_PAD - E
    src = jnp.concatenate([ei[0], jnp.zeros((npad,), jnp.int32)])
    dst = jnp.concatenate([ei[1], jnp.full((npad,), PAD_DST, jnp.int32)])
    dst2d = dst.reshape(E_PAD // CHUNK, CHUNK)
    dst3d = dst.reshape(NT, 1, CHUNK)

    deg = _sc_degree(dst2d)[:, :N].reshape(CORES, N, 1)
    hp1 = _tc_first(node_features, W1, deg)
    acc1 = _sc_message(hp1, src, dst3d)
    hp2 = _tc_mid(acc1, hp1, deg, b1.reshape(1, D), W2)
    acc2 = _sc_message(hp2, src, dst3d)
    return _tc_last(acc2, hp2, deg, b2.reshape(1, D),
                    W_fc[:D], W_fc[D:], b_fc.reshape(1, D))


# speedup vs baseline: 1.2408x; 1.0600x over previous
import functools

import jax
import jax.numpy as jnp
from jax import lax
from jax.experimental import pallas as pl
from jax.experimental.pallas import tpu as pltpu
from jax.experimental.pallas import tpu_sc as plsc

N = 10000
E = 320000
D = 128
TILES = 16
CORES = 2
CHUNK = 128
NCHUNK = 80
NC0 = 80
NC1 = 80
NT = (NC0 + NC1) * TILES
EPT = NCHUNK * CHUNK
E_PAD = EPT * TILES * CORES
N_ACC = 10112
N_DEG = 10240
PAD_DST = 10008
ROWS_T = 624
ROWS_LAST = N - 15 * ROWS_T
ACC_T = N_ACC // TILES
DEG_T = N_DEG // TILES

_MESH = plsc.VectorSubcoreMesh(core_axis_name="c", subcore_axis_name="s")


@functools.partial(
    pl.kernel,
    out_type=jax.ShapeDtypeStruct((CORES, N_DEG), jnp.float32),
    mesh=_MESH,
    scratch_types=[
        pltpu.VMEM((NCHUNK, CHUNK), jnp.int32),
        pltpu.VMEM((CHUNK,), jnp.float32),
        pltpu.VMEM((DEG_T,), jnp.float32),
        pltpu.VMEM_SHARED((N_DEG,), jnp.float32),
    ],
)
def _sc_degree(dst_hbm, deg_hbm, dst_v, ones_v, zer_v, deg_sh):
    c = lax.axis_index("c")
    s = lax.axis_index("s")
    for i in range(CHUNK // 16):
        ones_v[pl.ds(i * 16, 16)] = jnp.ones((16,), jnp.float32)
    def _z(i, _):
        zer_v[pl.ds(i * 16, 16)] = jnp.zeros((16,), jnp.float32)
        return 0
    lax.fori_loop(0, DEG_T // 16, _z, 0)
    pltpu.sync_copy(zer_v, deg_sh.at[pl.ds(s * DEG_T, DEG_T)])
    pltpu.sync_copy(
        dst_hbm.at[pl.ds((c * TILES + s) * NCHUNK, NCHUNK), :], dst_v)
    plsc.subcore_barrier()

    def _body(j, _):
        pltpu.sync_copy(ones_v, deg_sh.at[dst_v.at[j]], add=True)
        return 0
    lax.fori_loop(0, NCHUNK, _body, 0)
    plsc.subcore_barrier()
    pltpu.sync_copy(deg_sh.at[pl.ds(s * DEG_T, DEG_T)],
                    deg_hbm.at[c, pl.ds(s * DEG_T, DEG_T)])


@functools.partial(
    pl.kernel,
    out_type=jax.ShapeDtypeStruct((CORES, N, D), jnp.float32),
    mesh=_MESH,
    scratch_types=[
        pltpu.VMEM((CHUNK,), jnp.int32),
        pltpu.VMEM((CHUNK,), jnp.int32),
        pltpu.VMEM((CHUNK,), jnp.int32),
        pltpu.VMEM((1, CHUNK), jnp.int32),
        pltpu.VMEM((1, CHUNK), jnp.int32),
        pltpu.VMEM((1, CHUNK), jnp.int32),
        pltpu.VMEM((CHUNK, D), jnp.float32),
        pltpu.VMEM((CHUNK, D), jnp.float32),
        pltpu.VMEM((CHUNK, D), jnp.float32),
        pltpu.VMEM_SHARED((N_ACC, D), jnp.float32),
        pltpu.SemaphoreType.DMA,
        pltpu.SemaphoreType.DMA,
        pltpu.SemaphoreType.DMA,
    ],
)
def _sc_message(hp_hbm, src_hbm, dst_hbm, zeros_hbm, out_hbm,
                sidx0_v, sidx1_v, sidx2_v, didx0_v, didx1_v, didx2_v,
                rows0_v, rows1_v, rows2_v, acc_sh, sem_g, sem_i, sem_s):
    c = lax.axis_index("c")
    s = lax.axis_index("s")
    pltpu.sync_copy(zeros_hbm.at[pl.ds(s * ACC_T, ACC_T), :],
                    acc_sh.at[pl.ds(s * ACC_T, ACC_T), :])
    plsc.subcore_barrier()

    def _fetch_idx(t, ibuf, dbuf):
        pltpu.async_copy(src_hbm.at[pl.ds(t * CHUNK, CHUNK)], ibuf, sem_i)
        pltpu.async_copy(dst_hbm.at[t], dbuf, sem_i)

    def _iwait(ibuf, dbuf):
        pltpu.make_async_copy(src_hbm.at[pl.ds(0, CHUNK)], ibuf, sem_i).wait()
        pltpu.make_async_copy(dst_hbm.at[0], dbuf, sem_i).wait()

    def _gather(ibuf, buf):
        pltpu.async_copy(hp_hbm.at[ibuf], buf, sem_g)

    def _gwait(ibuf, buf):
        pltpu.make_async_copy(hp_hbm.at[ibuf], buf, sem_g).wait()

    def _scatter(buf, dbuf):
        pltpu.async_copy(buf, acc_sh.at[dbuf.at[0]], sem_s, add=True)

    def _swait():
        pltpu.make_async_copy(rows0_v, acc_sh.at[didx0_v.at[0]],
                              sem_s).wait()

    bufs = [(sidx0_v, didx0_v, rows0_v),
            (sidx1_v, didx1_v, rows1_v),
            (sidx2_v, didx2_v, rows2_v)]

    def _run(nck, t0):
        _fetch_idx(t0, bufs[0][0], bufs[0][1])
        _iwait(bufs[0][0], bufs[0][1])
        _gather(bufs[0][0], bufs[0][2])
        _fetch_idx(t0 + 1, bufs[1][0], bufs[1][1])

        def _body(g, _):
            def _step(cur, nxt, nxt2):
                _gwait(cur[0], cur[2])

                @pl.when(g + 1 < nck)
                def _launch_next():
                    _iwait(nxt[0], nxt[1])

                    @pl.when(g >= 2)
                    def _w():
                        _swait()
                    _gather(nxt[0], nxt[2])

                @pl.when(g + 2 < nck)
                def _prefetch_idx():
                    _fetch_idx(t0 + g + 2, nxt2[0], nxt2[1])
                _scatter(cur[2], cur[1])

            @pl.when(g % 3 == 0)
            def _r0():
                _step(bufs[0], bufs[1], bufs[2])

            @pl.when(g % 3 == 1)
            def _r1():
                _step(bufs[1], bufs[2], bufs[0])

            @pl.when(g % 3 == 2)
            def _r2():
                _step(bufs[2], bufs[0], bufs[1])
            return 0
        lax.fori_loop(0, nck, _body, 0)
        _swait()
        _swait()
        _swait()

    @pl.when(c == 0)
    def _core0():
        _run(NC0, s * NC0)

    @pl.when(c == 1)
    def _core1():
        _run(NC1, TILES * NC0 + s * NC1)
    plsc.subcore_barrier()

    @pl.when(s < 15)
    def _drain():
        pltpu.sync_copy(acc_sh.at[pl.ds(s * ROWS_T, ROWS_T), :],
                        out_hbm.at[c, pl.ds(s * ROWS_T, ROWS_T), :])

    @pl.when(s == 15)
    def _drain_last():
        pltpu.sync_copy(acc_sh.at[pl.ds(15 * ROWS_T, ROWS_LAST), :],
                        out_hbm.at[c, pl.ds(15 * ROWS_T, ROWS_LAST), :])


_BM = 1000


def _tc_first(x, W1, deg):
    def body(x_ref, w_ref, d_ref, o_ref):
        dinv = lax.rsqrt(d_ref[0] + d_ref[1] + 1.0)
        o_ref[...] = jnp.dot(x_ref[...], w_ref[...],
                             preferred_element_type=jnp.float32) * dinv
    return pl.pallas_call(
        body,
        grid=(N // _BM,),
        in_specs=[
            pl.BlockSpec((_BM, D), lambda i: (i, 0)),
            pl.BlockSpec((D, D), lambda i: (0, 0)),
            pl.BlockSpec((CORES, _BM, 1), lambda i: (0, i, 0)),
        ],
        out_specs=pl.BlockSpec((_BM, D), lambda i: (i, 0)),
        out_shape=jax.ShapeDtypeStruct((N, D), jnp.float32),
    )(x, W1, deg)


def _tc_mid(acc, hp, deg, b, W2):
    def body(a_ref, h_ref, d_ref, b_ref, w_ref, o_ref):
        dinv = lax.rsqrt(d_ref[0] + d_ref[1] + 1.0)
        tot = a_ref[0] + a_ref[1] + h_ref[...]
        x2 = jnp.maximum(tot * dinv + b_ref[...], 0.0)
        o_ref[...] = jnp.dot(x2, w_ref[...],
                             preferred_element_type=jnp.float32) * dinv
    return pl.pallas_call(
        body,
        grid=(N // _BM,),
        in_specs=[
            pl.BlockSpec((CORES, _BM, D), lambda i: (0, i, 0)),
            pl.BlockSpec((_BM, D), lambda i: (i, 0)),
            pl.BlockSpec((CORES, _BM, 1), lambda i: (0, i, 0)),
            pl.BlockSpec((1, D), lambda i: (0, 0)),
            pl.BlockSpec((D, D), lambda i: (0, 0)),
        ],
        out_specs=pl.BlockSpec((_BM, D), lambda i: (i, 0)),
        out_shape=jax.ShapeDtypeStruct((N, D), jnp.float32),
    )(acc, hp, deg, b, W2)


def _tc_last(acc, hp, deg, b, Wfm, Wfx, bfc):
    def body(a_ref, h_ref, d_ref, b_ref, wm_ref, wx_ref, bf_ref, o_ref,
             sum_ref, max_ref):
        i = pl.program_id(0)
        dinv = lax.rsqrt(d_ref[0] + d_ref[1] + 1.0)
        tot = a_ref[0] + a_ref[1] + h_ref[...]
        x3 = jnp.maximum(tot * dinv + b_ref[...], 0.0)
        bsum = jnp.sum(x3, axis=0, keepdims=True)
        bmax = jnp.max(x3, axis=0, keepdims=True)

        @pl.when(i == 0)
        def _init():
            sum_ref[...] = bsum
            max_ref[...] = bmax

        @pl.when(i > 0)
        def _accum():
            sum_ref[...] += bsum
            max_ref[...] = jnp.maximum(max_ref[...], bmax)

        @pl.when(i == N // _BM - 1)
        def _final():
            mean = sum_ref[...] * (1.0 / N)
            o_ref[...] = (jnp.dot(mean, wm_ref[...],
                                  preferred_element_type=jnp.float32)
                          + jnp.dot(max_ref[...], wx_ref[...],
                                    preferred_element_type=jnp.float32)
                          + bf_ref[...])
    return pl.pallas_call(
        body,
        grid=(N // _BM,),
        in_specs=[
            pl.BlockSpec((CORES, _BM, D), lambda i: (0, i, 0)),
            pl.BlockSpec((_BM, D), lambda i: (i, 0)),
            pl.BlockSpec((CORES, _BM, 1), lambda i: (0, i, 0)),
            pl.BlockSpec((1, D), lambda i: (0, 0)),
            pl.BlockSpec((D, D), lambda i: (0, 0)),
            pl.BlockSpec((D, D), lambda i: (0, 0)),
            pl.BlockSpec((1, D), lambda i: (0, 0)),
        ],
        out_specs=pl.BlockSpec((1, D), lambda i: (0, 0)),
        out_shape=jax.ShapeDtypeStruct((1, D), jnp.float32),
        scratch_shapes=[
            pltpu.VMEM((1, D), jnp.float32),
            pltpu.VMEM((1, D), jnp.float32),
        ],
    )(acc, hp, deg, b, Wfm, Wfx, bfc)


def kernel(node_features, edge_index, W1, b1, W2, b2, W_fc, b_fc):
    ei = edge_index.astype(jnp.int32)
    npad = E_PAD - E
    src = jnp.concatenate([ei[0], jnp.zeros((npad,), jnp.int32)])
    dst = jnp.concatenate([ei[1], jnp.full((npad,), PAD_DST, jnp.int32)])
    dst2d = dst.reshape(E_PAD // CHUNK, CHUNK)
    dst3d = dst.reshape(NT, 1, CHUNK)

    zeros = jnp.zeros((N_ACC, D), jnp.float32)

    deg = _sc_degree(dst2d)[:, :N].reshape(CORES, N, 1)
    hp1 = _tc_first(node_features, W1, deg)
    acc1 = _sc_message(hp1, src, dst3d, zeros)
    hp2 = _tc_mid(acc1, hp1, deg, b1.reshape(1, D), W2)
    acc2 = _sc_message(hp2, src, dst3d, zeros)
    return _tc_last(acc2, hp2, deg, b2.reshape(1, D),
                    W_fc[:D], W_fc[D:], b_fc.reshape(1, D))


# speedup vs baseline: 1.3958x; 1.1249x over previous
import functools

import jax
import jax.numpy as jnp
from jax import lax
from jax.experimental import pallas as pl
from jax.experimental.pallas import tpu as pltpu
from jax.experimental.pallas import tpu_sc as plsc

N = 10000
E = 320000
D = 128
TILES = 16
CORES = 2
CHUNK = 128
NCHUNK = 80
NC0 = 80
NC1 = 80
NT = (NC0 + NC1) * TILES
EPT = NCHUNK * CHUNK
E_PAD = EPT * TILES * CORES
N_ACC = 10112
N_DEG = 10240
PAD_DST = 10008
ROWS_T = 624
ROWS_LAST = N - 15 * ROWS_T
ACC_T = N_ACC // TILES
DEG_T = N_DEG // TILES

_MESH = plsc.VectorSubcoreMesh(core_axis_name="c", subcore_axis_name="s")


@functools.partial(
    pl.kernel,
    out_type=jax.ShapeDtypeStruct((CORES, N_DEG), jnp.float32),
    mesh=_MESH,
    scratch_types=[
        pltpu.VMEM((NCHUNK, CHUNK), jnp.int32),
        pltpu.VMEM((CHUNK,), jnp.float32),
        pltpu.VMEM((DEG_T,), jnp.float32),
        pltpu.VMEM_SHARED((N_DEG,), jnp.float32),
    ],
)
def _sc_degree(dst_hbm, deg_hbm, dst_v, ones_v, zer_v, deg_sh):
    c = lax.axis_index("c")
    s = lax.axis_index("s")
    for i in range(CHUNK // 16):
        ones_v[pl.ds(i * 16, 16)] = jnp.ones((16,), jnp.float32)
    def _z(i, _):
        zer_v[pl.ds(i * 16, 16)] = jnp.zeros((16,), jnp.float32)
        return 0
    lax.fori_loop(0, DEG_T // 16, _z, 0)
    pltpu.sync_copy(zer_v, deg_sh.at[pl.ds(s * DEG_T, DEG_T)])
    pltpu.sync_copy(
        dst_hbm.at[pl.ds((c * TILES + s) * NCHUNK, NCHUNK), :], dst_v)
    plsc.subcore_barrier()

    def _body(j, _):
        pltpu.sync_copy(ones_v, deg_sh.at[dst_v.at[j]], add=True)
        return 0
    lax.fori_loop(0, NCHUNK, _body, 0)
    plsc.subcore_barrier()
    pltpu.sync_copy(deg_sh.at[pl.ds(s * DEG_T, DEG_T)],
                    deg_hbm.at[c, pl.ds(s * DEG_T, DEG_T)])


@functools.partial(
    pl.kernel,
    out_type=jax.ShapeDtypeStruct((CORES, N, D), jnp.float32),
    mesh=_MESH,
    scratch_types=[
        pltpu.VMEM((CHUNK,), jnp.int32),
        pltpu.VMEM((CHUNK,), jnp.int32),
        pltpu.VMEM((CHUNK,), jnp.int32),
        pltpu.VMEM((1, CHUNK), jnp.int32),
        pltpu.VMEM((1, CHUNK), jnp.int32),
        pltpu.VMEM((1, CHUNK), jnp.int32),
        pltpu.VMEM((CHUNK, D), jnp.float32),
        pltpu.VMEM((CHUNK, D), jnp.float32),
        pltpu.VMEM((CHUNK, D), jnp.float32),
        pltpu.VMEM_SHARED((N_ACC, D), jnp.float32),
        pltpu.SemaphoreType.DMA,
        pltpu.SemaphoreType.DMA,
        pltpu.SemaphoreType.DMA,
    ],
)
def _sc_message(hp_hbm, src_hbm, dst_hbm, zeros_hbm, out_hbm,
                sidx0_v, sidx1_v, sidx2_v, didx0_v, didx1_v, didx2_v,
                rows0_v, rows1_v, rows2_v, acc_sh, sem_g, sem_i, sem_s):
    c = lax.axis_index("c")
    s = lax.axis_index("s")
    pltpu.sync_copy(zeros_hbm.at[pl.ds(s * ACC_T, ACC_T), :],
                    acc_sh.at[pl.ds(s * ACC_T, ACC_T), :])
    plsc.subcore_barrier()

    def _fetch_idx(t, ibuf, dbuf):
        pltpu.async_copy(src_hbm.at[c, pl.ds(t * CHUNK, CHUNK)], ibuf, sem_i)
        pltpu.async_copy(dst_hbm.at[t], dbuf, sem_i)

    def _iwait(ibuf, dbuf):
        pltpu.make_async_copy(src_hbm.at[0, pl.ds(0, CHUNK)],
                              ibuf, sem_i).wait()
        pltpu.make_async_copy(dst_hbm.at[0], dbuf, sem_i).wait()

    def _gather(ibuf, buf):
        pltpu.async_copy(hp_hbm.at[ibuf], buf, sem_g)

    def _gwait(ibuf, buf):
        pltpu.make_async_copy(hp_hbm.at[ibuf], buf, sem_g).wait()

    def _scatter(buf, dbuf):
        pltpu.async_copy(buf, acc_sh.at[dbuf.at[0]], sem_s, add=True)

    def _swait():
        pltpu.make_async_copy(rows0_v, acc_sh.at[didx0_v.at[0]],
                              sem_s).wait()

    bufs = [(sidx0_v, didx0_v, rows0_v),
            (sidx1_v, didx1_v, rows1_v),
            (sidx2_v, didx2_v, rows2_v)]

    def _run(nck, t0):
        _fetch_idx(t0, bufs[0][0], bufs[0][1])
        _iwait(bufs[0][0], bufs[0][1])
        _gather(bufs[0][0], bufs[0][2])
        _fetch_idx(t0 + 1, bufs[1][0], bufs[1][1])

        def _body(g, _):
            def _step(cur, nxt, nxt2):
                _gwait(cur[0], cur[2])

                @pl.when(g + 1 < nck)
                def _launch_next():
                    _iwait(nxt[0], nxt[1])

                    @pl.when(g >= 2)
                    def _w():
                        _swait()
                    _gather(nxt[0], nxt[2])

                @pl.when(g + 2 < nck)
                def _prefetch_idx():
                    _fetch_idx(t0 + g + 2, nxt2[0], nxt2[1])
                _scatter(cur[2], cur[1])

            @pl.when(g % 3 == 0)
            def _r0():
                _step(bufs[0], bufs[1], bufs[2])

            @pl.when(g % 3 == 1)
            def _r1():
                _step(bufs[1], bufs[2], bufs[0])

            @pl.when(g % 3 == 2)
            def _r2():
                _step(bufs[2], bufs[0], bufs[1])
            return 0
        lax.fori_loop(0, nck, _body, 0)
        _swait()
        _swait()
        _swait()

    @pl.when(c == 0)
    def _core0():
        _run(NC0, s * NC0)

    @pl.when(c == 1)
    def _core1():
        _run(NC1, TILES * NC0 + s * NC1)
    plsc.subcore_barrier()

    @pl.when(s < 15)
    def _drain():
        pltpu.sync_copy(acc_sh.at[pl.ds(s * ROWS_T, ROWS_T), :],
                        out_hbm.at[c, pl.ds(s * ROWS_T, ROWS_T), :])

    @pl.when(s == 15)
    def _drain_last():
        pltpu.sync_copy(acc_sh.at[pl.ds(15 * ROWS_T, ROWS_LAST), :],
                        out_hbm.at[c, pl.ds(15 * ROWS_T, ROWS_LAST), :])


_BM = 1000


def _tc_first(x, W1, deg):
    def body(x_ref, w_ref, d_ref, o_ref):
        dinv = lax.rsqrt(d_ref[0] + d_ref[1] + 1.0)
        h = jnp.dot(x_ref[...], w_ref[...],
                    preferred_element_type=jnp.float32) * dinv
        o_ref[0] = h
        o_ref[1] = h
    return pl.pallas_call(
        body,
        grid=(N // _BM,),
        in_specs=[
            pl.BlockSpec((_BM, D), lambda i: (i, 0)),
            pl.BlockSpec((D, D), lambda i: (0, 0)),
            pl.BlockSpec((CORES, _BM, 1), lambda i: (0, i, 0)),
        ],
        out_specs=pl.BlockSpec((CORES, _BM, D), lambda i: (0, i, 0)),
        out_shape=jax.ShapeDtypeStruct((CORES, N, D), jnp.float32),
    )(x, W1, deg)


def _tc_mid(acc, hp, deg, b, W2):
    def body(a_ref, h_ref, d_ref, b_ref, w_ref, o_ref):
        dinv = lax.rsqrt(d_ref[0] + d_ref[1] + 1.0)
        tot = a_ref[0] + a_ref[1] + h_ref[0]
        x2 = jnp.maximum(tot * dinv + b_ref[...], 0.0)
        h = jnp.dot(x2, w_ref[...],
                    preferred_element_type=jnp.float32) * dinv
        o_ref[0] = h
        o_ref[1] = h
    return pl.pallas_call(
        body,
        grid=(N // _BM,),
        in_specs=[
            pl.BlockSpec((CORES, _BM, D), lambda i: (0, i, 0)),
            pl.BlockSpec((CORES, _BM, D), lambda i: (0, i, 0)),
            pl.BlockSpec((CORES, _BM, 1), lambda i: (0, i, 0)),
            pl.BlockSpec((1, D), lambda i: (0, 0)),
            pl.BlockSpec((D, D), lambda i: (0, 0)),
        ],
        out_specs=pl.BlockSpec((CORES, _BM, D), lambda i: (0, i, 0)),
        out_shape=jax.ShapeDtypeStruct((CORES, N, D), jnp.float32),
    )(acc, hp, deg, b, W2)


def _tc_last(acc, hp, deg, b, Wfm, Wfx, bfc):
    def body(a_ref, h_ref, d_ref, b_ref, wm_ref, wx_ref, bf_ref, o_ref,
             sum_ref, max_ref):
        i = pl.program_id(0)
        dinv = lax.rsqrt(d_ref[0] + d_ref[1] + 1.0)
        tot = a_ref[0] + a_ref[1] + h_ref[0]
        x3 = jnp.maximum(tot * dinv + b_ref[...], 0.0)
        bsum = jnp.sum(x3, axis=0, keepdims=True)
        bmax = jnp.max(x3, axis=0, keepdims=True)

        @pl.when(i == 0)
        def _init():
            sum_ref[...] = bsum
            max_ref[...] = bmax

        @pl.when(i > 0)
        def _accum():
            sum_ref[...] += bsum
            max_ref[...] = jnp.maximum(max_ref[...], bmax)

        @pl.when(i == N // _BM - 1)
        def _final():
            mean = sum_ref[...] * (1.0 / N)
            o_ref[...] = (jnp.dot(mean, wm_ref[...],
                                  preferred_element_type=jnp.float32)
                          + jnp.dot(max_ref[...], wx_ref[...],
                                    preferred_element_type=jnp.float32)
                          + bf_ref[...])
    return pl.pallas_call(
        body,
        grid=(N // _BM,),
        in_specs=[
            pl.BlockSpec((CORES, _BM, D), lambda i: (0, i, 0)),
            pl.BlockSpec((CORES, _BM, D), lambda i: (0, i, 0)),
            pl.BlockSpec((CORES, _BM, 1), lambda i: (0, i, 0)),
            pl.BlockSpec((1, D), lambda i: (0, 0)),
            pl.BlockSpec((D, D), lambda i: (0, 0)),
            pl.BlockSpec((D, D), lambda i: (0, 0)),
            pl.BlockSpec((1, D), lambda i: (0, 0)),
        ],
        out_specs=pl.BlockSpec((1, D), lambda i: (0, 0)),
        out_shape=jax.ShapeDtypeStruct((1, D), jnp.float32),
        scratch_shapes=[
            pltpu.VMEM((1, D), jnp.float32),
            pltpu.VMEM((1, D), jnp.float32),
        ],
    )(acc, hp, deg, b, Wfm, Wfx, bfc)


def kernel(node_features, edge_index, W1, b1, W2, b2, W_fc, b_fc):
    ei = edge_index.astype(jnp.int32)
    npad = E_PAD - E
    src = jnp.concatenate([ei[0], jnp.zeros((npad,), jnp.int32)])
    dst = jnp.concatenate([ei[1], jnp.full((npad,), PAD_DST, jnp.int32)])
    dst2d = dst.reshape(E_PAD // CHUNK, CHUNK)
    dst3d = dst.reshape(NT, 1, CHUNK)

    zeros = jnp.zeros((N_ACC, D), jnp.float32)
    src2 = jnp.stack([src, src + N])

    deg = _sc_degree(dst2d)[:, :N].reshape(CORES, N, 1)
    hp1 = _tc_first(node_features, W1, deg)
    acc1 = _sc_message(hp1.reshape(CORES * N, D), src2, dst3d, zeros)
    hp2 = _tc_mid(acc1, hp1, deg, b1.reshape(1, D), W2)
    acc2 = _sc_message(hp2.reshape(CORES * N, D), src2, dst3d, zeros)
    return _tc_last(acc2, hp2, deg, b2.reshape(1, D),
                    W_fc[:D], W_fc[D:], b_fc.reshape(1, D))


# speedup vs baseline: 1.5255x; 1.0929x over previous
import functools

import jax
import jax.numpy as jnp
from jax import lax
from jax.experimental import pallas as pl
from jax.experimental.pallas import tpu as pltpu
from jax.experimental.pallas import tpu_sc as plsc

N = 10000
E = 320000
D = 128
TILES = 16
CORES = 2
CHUNK = 128
NCHUNK = 80
NC0 = 120
NC1 = 40
NT = (NC0 + NC1) * TILES
EPT = NCHUNK * CHUNK
E_PAD = EPT * TILES * CORES
N_ACC = 10112
N_DEG = 10240
PAD_DST = 10008
ROWS_T = 624
ROWS_LAST = N - 15 * ROWS_T
ACC_T = N_ACC // TILES
DEG_T = N_DEG // TILES

_MESH = plsc.VectorSubcoreMesh(core_axis_name="c", subcore_axis_name="s")


@functools.partial(
    pl.kernel,
    out_type=jax.ShapeDtypeStruct((CORES, N_DEG), jnp.float32),
    mesh=_MESH,
    scratch_types=[
        pltpu.VMEM((NCHUNK, CHUNK), jnp.int32),
        pltpu.VMEM((CHUNK,), jnp.float32),
        pltpu.VMEM((DEG_T,), jnp.float32),
        pltpu.VMEM_SHARED((N_DEG,), jnp.float32),
    ],
)
def _sc_degree(dst_hbm, deg_hbm, dst_v, ones_v, zer_v, deg_sh):
    c = lax.axis_index("c")
    s = lax.axis_index("s")
    for i in range(CHUNK // 16):
        ones_v[pl.ds(i * 16, 16)] = jnp.ones((16,), jnp.float32)
    def _z(i, _):
        zer_v[pl.ds(i * 16, 16)] = jnp.zeros((16,), jnp.float32)
        return 0
    lax.fori_loop(0, DEG_T // 16, _z, 0)
    pltpu.sync_copy(zer_v, deg_sh.at[pl.ds(s * DEG_T, DEG_T)])
    pltpu.sync_copy(
        dst_hbm.at[pl.ds((c * TILES + s) * NCHUNK, NCHUNK), :], dst_v)
    plsc.subcore_barrier()

    def _body(j, _):
        pltpu.sync_copy(ones_v, deg_sh.at[dst_v.at[j]], add=True)
        return 0
    lax.fori_loop(0, NCHUNK, _body, 0)
    plsc.subcore_barrier()
    pltpu.sync_copy(deg_sh.at[pl.ds(s * DEG_T, DEG_T)],
                    deg_hbm.at[c, pl.ds(s * DEG_T, DEG_T)])


@functools.partial(
    pl.kernel,
    out_type=jax.ShapeDtypeStruct((CORES, N, D), jnp.float32),
    mesh=_MESH,
    scratch_types=[
        pltpu.VMEM((CHUNK,), jnp.int32),
        pltpu.VMEM((CHUNK,), jnp.int32),
        pltpu.VMEM((CHUNK,), jnp.int32),
        pltpu.VMEM((1, CHUNK), jnp.int32),
        pltpu.VMEM((1, CHUNK), jnp.int32),
        pltpu.VMEM((1, CHUNK), jnp.int32),
        pltpu.VMEM((CHUNK, D), jnp.float32),
        pltpu.VMEM((CHUNK, D), jnp.float32),
        pltpu.VMEM((CHUNK, D), jnp.float32),
        pltpu.VMEM_SHARED((N_ACC, D), jnp.float32),
        pltpu.SemaphoreType.DMA,
        pltpu.SemaphoreType.DMA,
        pltpu.SemaphoreType.DMA,
    ],
)
def _sc_message(hp_hbm, src_hbm, dst_hbm, zeros_hbm, out_hbm,
                sidx0_v, sidx1_v, sidx2_v, didx0_v, didx1_v, didx2_v,
                rows0_v, rows1_v, rows2_v, acc_sh, sem_g, sem_i, sem_s):
    c = lax.axis_index("c")
    s = lax.axis_index("s")
    pltpu.sync_copy(zeros_hbm.at[pl.ds(s * ACC_T, ACC_T), :],
                    acc_sh.at[pl.ds(s * ACC_T, ACC_T), :])
    plsc.subcore_barrier()

    def _fetch_idx(t, ibuf, dbuf):
        pltpu.async_copy(src_hbm.at[c, pl.ds(t * CHUNK, CHUNK)], ibuf, sem_i)
        pltpu.async_copy(dst_hbm.at[t], dbuf, sem_i)

    def _iwait(ibuf, dbuf):
        pltpu.make_async_copy(src_hbm.at[0, pl.ds(0, CHUNK)],
                              ibuf, sem_i).wait()
        pltpu.make_async_copy(dst_hbm.at[0], dbuf, sem_i).wait()

    def _gather(ibuf, buf):
        pltpu.async_copy(hp_hbm.at[ibuf], buf, sem_g)

    def _gwait(ibuf, buf):
        pltpu.make_async_copy(hp_hbm.at[ibuf], buf, sem_g).wait()

    def _scatter(buf, dbuf):
        pltpu.async_copy(buf, acc_sh.at[dbuf.at[0]], sem_s, add=True)

    def _swait():
        pltpu.make_async_copy(rows0_v, acc_sh.at[didx0_v.at[0]],
                              sem_s).wait()

    bufs = [(sidx0_v, didx0_v, rows0_v),
            (sidx1_v, didx1_v, rows1_v),
            (sidx2_v, didx2_v, rows2_v)]

    def _run(nck, t0):
        _fetch_idx(t0, bufs[0][0], bufs[0][1])
        _iwait(bufs[0][0], bufs[0][1])
        _gather(bufs[0][0], bufs[0][2])
        _fetch_idx(t0 + 1, bufs[1][0], bufs[1][1])

        def _body(g, _):
            def _step(cur, nxt, nxt2):
                _gwait(cur[0], cur[2])

                @pl.when(g + 1 < nck)
                def _launch_next():
                    _iwait(nxt[0], nxt[1])

                    @pl.when(g >= 2)
                    def _w():
                        _swait()
                    _gather(nxt[0], nxt[2])

                @pl.when(g + 2 < nck)
                def _prefetch_idx():
                    _fetch_idx(t0 + g + 2, nxt2[0], nxt2[1])
                _scatter(cur[2], cur[1])

            @pl.when(g % 3 == 0)
            def _r0():
                _step(bufs[0], bufs[1], bufs[2])

            @pl.when(g % 3 == 1)
            def _r1():
                _step(bufs[1], bufs[2], bufs[0])

            @pl.when(g % 3 == 2)
            def _r2():
                _step(bufs[2], bufs[0], bufs[1])
            return 0
        lax.fori_loop(0, nck, _body, 0)
        _swait()
        _swait()
        _swait()

    @pl.when(c == 0)
    def _core0():
        _run(NC0, s * NC0)

    @pl.when(c == 1)
    def _core1():
        _run(NC1, TILES * NC0 + s * NC1)
    plsc.subcore_barrier()

    @pl.when(s < 15)
    def _drain():
        pltpu.sync_copy(acc_sh.at[pl.ds(s * ROWS_T, ROWS_T), :],
                        out_hbm.at[c, pl.ds(s * ROWS_T, ROWS_T), :])

    @pl.when(s == 15)
    def _drain_last():
        pltpu.sync_copy(acc_sh.at[pl.ds(15 * ROWS_T, ROWS_LAST), :],
                        out_hbm.at[c, pl.ds(15 * ROWS_T, ROWS_LAST), :])


_BM = 1000


def _tc_first(x, W1, deg):
    def body(x_ref, w_ref, d_ref, o_ref):
        dinv = lax.rsqrt(d_ref[0] + d_ref[1] + 1.0)
        h = jnp.dot(x_ref[...], w_ref[...],
                    preferred_element_type=jnp.float32) * dinv
        o_ref[0] = h
        o_ref[1] = h
    return pl.pallas_call(
        body,
        grid=(N // _BM,),
        in_specs=[
            pl.BlockSpec((_BM, D), lambda i: (i, 0)),
            pl.BlockSpec((D, D), lambda i: (0, 0)),
            pl.BlockSpec((CORES, _BM, 1), lambda i: (0, i, 0)),
        ],
        out_specs=pl.BlockSpec((CORES, _BM, D), lambda i: (0, i, 0)),
        out_shape=jax.ShapeDtypeStruct((CORES, N, D), jnp.float32),
    )(x, W1, deg)


def _tc_mid(acc, hp, deg, b, W2):
    def body(a_ref, h_ref, d_ref, b_ref, w_ref, o_ref):
        dinv = lax.rsqrt(d_ref[0] + d_ref[1] + 1.0)
        tot = a_ref[0] + a_ref[1] + h_ref[0]
        x2 = jnp.maximum(tot * dinv + b_ref[...], 0.0)
        h = jnp.dot(x2, w_ref[...],
                    preferred_element_type=jnp.float32) * dinv
        o_ref[0] = h
        o_ref[1] = h
    return pl.pallas_call(
        body,
        grid=(N // _BM,),
        in_specs=[
            pl.BlockSpec((CORES, _BM, D), lambda i: (0, i, 0)),
            pl.BlockSpec((CORES, _BM, D), lambda i: (0, i, 0)),
            pl.BlockSpec((CORES, _BM, 1), lambda i: (0, i, 0)),
            pl.BlockSpec((1, D), lambda i: (0, 0)),
            pl.BlockSpec((D, D), lambda i: (0, 0)),
        ],
        out_specs=pl.BlockSpec((CORES, _BM, D), lambda i: (0, i, 0)),
        out_shape=jax.ShapeDtypeStruct((CORES, N, D), jnp.float32),
    )(acc, hp, deg, b, W2)


def _tc_last(acc, hp, deg, b, Wfm, Wfx, bfc):
    def body(a_ref, h_ref, d_ref, b_ref, wm_ref, wx_ref, bf_ref, o_ref,
             sum_ref, max_ref):
        i = pl.program_id(0)
        dinv = lax.rsqrt(d_ref[0] + d_ref[1] + 1.0)
        tot = a_ref[0] + a_ref[1] + h_ref[0]
        x3 = jnp.maximum(tot * dinv + b_ref[...], 0.0)
        bsum = jnp.sum(x3, axis=0, keepdims=True)
        bmax = jnp.max(x3, axis=0, keepdims=True)

        @pl.when(i == 0)
        def _init():
            sum_ref[...] = bsum
            max_ref[...] = bmax

        @pl.when(i > 0)
        def _accum():
            sum_ref[...] += bsum
            max_ref[...] = jnp.maximum(max_ref[...], bmax)

        @pl.when(i == N // _BM - 1)
        def _final():
            mean = sum_ref[...] * (1.0 / N)
            o_ref[...] = (jnp.dot(mean, wm_ref[...],
                                  preferred_element_type=jnp.float32)
                          + jnp.dot(max_ref[...], wx_ref[...],
                                    preferred_element_type=jnp.float32)
                          + bf_ref[...])
    return pl.pallas_call(
        body,
        grid=(N // _BM,),
        in_specs=[
            pl.BlockSpec((CORES, _BM, D), lambda i: (0, i, 0)),
            pl.BlockSpec((CORES, _BM, D), lambda i: (0, i, 0)),
            pl.BlockSpec((CORES, _BM, 1), lambda i: (0, i, 0)),
            pl.BlockSpec((1, D), lambda i: (0, 0)),
            pl.BlockSpec((D, D), lambda i: (0, 0)),
            pl.BlockSpec((D, D), lambda i: (0, 0)),
            pl.BlockSpec((1, D), lambda i: (0, 0)),
        ],
        out_specs=pl.BlockSpec((1, D), lambda i: (0, 0)),
        out_shape=jax.ShapeDtypeStruct((1, D), jnp.float32),
        scratch_shapes=[
            pltpu.VMEM((1, D), jnp.float32),
            pltpu.VMEM((1, D), jnp.float32),
        ],
    )(acc, hp, deg, b, Wfm, Wfx, bfc)


def kernel(node_features, edge_index, W1, b1, W2, b2, W_fc, b_fc):
    ei = edge_index.astype(jnp.int32)
    npad = E_PAD - E
    src = jnp.concatenate([ei[0], jnp.zeros((npad,), jnp.int32)])
    dst = jnp.concatenate([ei[1], jnp.full((npad,), PAD_DST, jnp.int32)])
    dst2d = dst.reshape(E_PAD // CHUNK, CHUNK)
    dst3d = dst.reshape(NT, 1, CHUNK)

    zeros = jnp.zeros((N_ACC, D), jnp.float32)
    src2 = jnp.stack([src, src + N])

    deg = _sc_degree(dst2d)[:, :N].reshape(CORES, N, 1)
    hp1 = _tc_first(node_features, W1, deg)
    acc1 = _sc_message(hp1.reshape(CORES * N, D), src2, dst3d, zeros)
    hp2 = _tc_mid(acc1, hp1, deg, b1.reshape(1, D), W2)
    acc2 = _sc_message(hp2.reshape(CORES * N, D), src2, dst3d, zeros)
    return _tc_last(acc2, hp2, deg, b2.reshape(1, D),
                    W_fc[:D], W_fc[D:], b_fc.reshape(1, D))


# speedup vs baseline: 1.5679x; 1.0278x over previous
import functools

import jax
import jax.numpy as jnp
from jax import lax
from jax.experimental import pallas as pl
from jax.experimental.pallas import tpu as pltpu
from jax.experimental.pallas import tpu_sc as plsc

N = 10000
E = 320000
D = 128
TILES = 16
CORES = 2
CHUNK = 128
NCHUNK = 80
NC0 = 136
NC1 = 24
NT = (NC0 + NC1) * TILES
EPT = NCHUNK * CHUNK
E_PAD = EPT * TILES * CORES
N_ACC = 10112
N_DEG = 10240
PAD_DST = 10008
ROWS_T = 624
ROWS_LAST = N - 15 * ROWS_T
ACC_T = N_ACC // TILES
DEG_T = N_DEG // TILES

_MESH = plsc.VectorSubcoreMesh(core_axis_name="c", subcore_axis_name="s")


@functools.partial(
    pl.kernel,
    out_type=jax.ShapeDtypeStruct((CORES, N_DEG), jnp.float32),
    mesh=_MESH,
    scratch_types=[
        pltpu.VMEM((NCHUNK, CHUNK), jnp.int32),
        pltpu.VMEM((CHUNK,), jnp.float32),
        pltpu.VMEM((DEG_T,), jnp.float32),
        pltpu.VMEM_SHARED((N_DEG,), jnp.float32),
    ],
)
def _sc_degree(dst_hbm, deg_hbm, dst_v, ones_v, zer_v, deg_sh):
    c = lax.axis_index("c")
    s = lax.axis_index("s")
    for i in range(CHUNK // 16):
        ones_v[pl.ds(i * 16, 16)] = jnp.ones((16,), jnp.float32)
    def _z(i, _):
        zer_v[pl.ds(i * 16, 16)] = jnp.zeros((16,), jnp.float32)
        return 0
    lax.fori_loop(0, DEG_T // 16, _z, 0)
    pltpu.sync_copy(zer_v, deg_sh.at[pl.ds(s * DEG_T, DEG_T)])
    pltpu.sync_copy(
        dst_hbm.at[pl.ds((c * TILES + s) * NCHUNK, NCHUNK), :], dst_v)
    plsc.subcore_barrier()

    def _body(j, _):
        pltpu.sync_copy(ones_v, deg_sh.at[dst_v.at[j]], add=True)
        return 0
    lax.fori_loop(0, NCHUNK, _body, 0)
    plsc.subcore_barrier()
    pltpu.sync_copy(deg_sh.at[pl.ds(s * DEG_T, DEG_T)],
                    deg_hbm.at[c, pl.ds(s * DEG_T, DEG_T)])


@functools.partial(
    pl.kernel,
    out_type=jax.ShapeDtypeStruct((CORES, N, D), jnp.float32),
    mesh=_MESH,
    scratch_types=[
        pltpu.VMEM((CHUNK,), jnp.int32),
        pltpu.VMEM((CHUNK,), jnp.int32),
        pltpu.VMEM((CHUNK,), jnp.int32),
        pltpu.VMEM((1, CHUNK), jnp.int32),
        pltpu.VMEM((1, CHUNK), jnp.int32),
        pltpu.VMEM((1, CHUNK), jnp.int32),
        pltpu.VMEM((CHUNK, D), jnp.float32),
        pltpu.VMEM((CHUNK, D), jnp.float32),
        pltpu.VMEM((CHUNK, D), jnp.float32),
        pltpu.VMEM_SHARED((N_ACC, D), jnp.float32),
        pltpu.SemaphoreType.DMA,
        pltpu.SemaphoreType.DMA,
        pltpu.SemaphoreType.DMA,
    ],
)
def _sc_message(hp_hbm, src_hbm, dst_hbm, zeros_hbm, out_hbm,
                sidx0_v, sidx1_v, sidx2_v, didx0_v, didx1_v, didx2_v,
                rows0_v, rows1_v, rows2_v, acc_sh, sem_g, sem_i, sem_s):
    c = lax.axis_index("c")
    s = lax.axis_index("s")
    pltpu.sync_copy(zeros_hbm.at[pl.ds(s * ACC_T, ACC_T), :],
                    acc_sh.at[pl.ds(s * ACC_T, ACC_T), :])
    plsc.subcore_barrier()

    def _fetch_idx(t, ibuf, dbuf):
        pltpu.async_copy(src_hbm.at[c, pl.ds(t * CHUNK, CHUNK)], ibuf, sem_i)
        pltpu.async_copy(dst_hbm.at[t], dbuf, sem_i)

    def _iwait(ibuf, dbuf):
        pltpu.make_async_copy(src_hbm.at[0, pl.ds(0, CHUNK)],
                              ibuf, sem_i).wait()
        pltpu.make_async_copy(dst_hbm.at[0], dbuf, sem_i).wait()

    def _gather(ibuf, buf):
        pltpu.async_copy(hp_hbm.at[ibuf], buf, sem_g)

    def _gwait(ibuf, buf):
        pltpu.make_async_copy(hp_hbm.at[ibuf], buf, sem_g).wait()

    def _scatter(buf, dbuf):
        pltpu.async_copy(buf, acc_sh.at[dbuf.at[0]], sem_s, add=True)

    def _swait():
        pltpu.make_async_copy(rows0_v, acc_sh.at[didx0_v.at[0]],
                              sem_s).wait()

    bufs = [(sidx0_v, didx0_v, rows0_v),
            (sidx1_v, didx1_v, rows1_v),
            (sidx2_v, didx2_v, rows2_v)]

    def _run(nck, t0):
        _fetch_idx(t0, bufs[0][0], bufs[0][1])
        _iwait(bufs[0][0], bufs[0][1])
        _gather(bufs[0][0], bufs[0][2])
        _fetch_idx(t0 + 1, bufs[1][0], bufs[1][1])

        def _body(g, _):
            def _step(cur, nxt, nxt2):
                _gwait(cur[0], cur[2])

                @pl.when(g + 1 < nck)
                def _launch_next():
                    _iwait(nxt[0], nxt[1])

                    @pl.when(g >= 2)
                    def _w():
                        _swait()
                    _gather(nxt[0], nxt[2])

                @pl.when(g + 2 < nck)
                def _prefetch_idx():
                    _fetch_idx(t0 + g + 2, nxt2[0], nxt2[1])
                _scatter(cur[2], cur[1])

            @pl.when(g % 3 == 0)
            def _r0():
                _step(bufs[0], bufs[1], bufs[2])

            @pl.when(g % 3 == 1)
            def _r1():
                _step(bufs[1], bufs[2], bufs[0])

            @pl.when(g % 3 == 2)
            def _r2():
                _step(bufs[2], bufs[0], bufs[1])
            return 0
        lax.fori_loop(0, nck, _body, 0)
        _swait()
        _swait()
        _swait()

    @pl.when(c == 0)
    def _core0():
        _run(NC0, s * NC0)

    @pl.when(c == 1)
    def _core1():
        _run(NC1, TILES * NC0 + s * NC1)
    plsc.subcore_barrier()

    @pl.when(s < 15)
    def _drain():
        pltpu.sync_copy(acc_sh.at[pl.ds(s * ROWS_T, ROWS_T), :],
                        out_hbm.at[c, pl.ds(s * ROWS_T, ROWS_T), :])

    @pl.when(s == 15)
    def _drain_last():
        pltpu.sync_copy(acc_sh.at[pl.ds(15 * ROWS_T, ROWS_LAST), :],
                        out_hbm.at[c, pl.ds(15 * ROWS_T, ROWS_LAST), :])


_BM = 1000


def _tc_first(x, W1, deg):
    def body(x_ref, w_ref, d_ref, o_ref):
        dinv = lax.rsqrt(d_ref[0] + d_ref[1] + 1.0)
        h = jnp.dot(x_ref[...], w_ref[...],
                    preferred_element_type=jnp.float32) * dinv
        o_ref[0] = h
        o_ref[1] = h
    return pl.pallas_call(
        body,
        grid=(N // _BM,),
        in_specs=[
            pl.BlockSpec((_BM, D), lambda i: (i, 0)),
            pl.BlockSpec((D, D), lambda i: (0, 0)),
            pl.BlockSpec((CORES, _BM, 1), lambda i: (0, i, 0)),
        ],
        out_specs=pl.BlockSpec((CORES, _BM, D), lambda i: (0, i, 0)),
        out_shape=jax.ShapeDtypeStruct((CORES, N, D), jnp.float32),
    )(x, W1, deg)


def _tc_mid(acc, hp, deg, b, W2):
    def body(a_ref, h_ref, d_ref, b_ref, w_ref, o_ref):
        dinv = lax.rsqrt(d_ref[0] + d_ref[1] + 1.0)
        tot = a_ref[0] + a_ref[1] + h_ref[0]
        x2 = jnp.maximum(tot * dinv + b_ref[...], 0.0)
        h = jnp.dot(x2, w_ref[...],
                    preferred_element_type=jnp.float32) * dinv
        o_ref[0] = h
        o_ref[1] = h
    return pl.pallas_call(
        body,
        grid=(N // _BM,),
        in_specs=[
            pl.BlockSpec((CORES, _BM, D), lambda i: (0, i, 0)),
            pl.BlockSpec((CORES, _BM, D), lambda i: (0, i, 0)),
            pl.BlockSpec((CORES, _BM, 1), lambda i: (0, i, 0)),
            pl.BlockSpec((1, D), lambda i: (0, 0)),
            pl.BlockSpec((D, D), lambda i: (0, 0)),
        ],
        out_specs=pl.BlockSpec((CORES, _BM, D), lambda i: (0, i, 0)),
        out_shape=jax.ShapeDtypeStruct((CORES, N, D), jnp.float32),
    )(acc, hp, deg, b, W2)


def _tc_last(acc, hp, deg, b, Wfm, Wfx, bfc):
    def body(a_ref, h_ref, d_ref, b_ref, wm_ref, wx_ref, bf_ref, o_ref,
             sum_ref, max_ref):
        i = pl.program_id(0)
        dinv = lax.rsqrt(d_ref[0] + d_ref[1] + 1.0)
        tot = a_ref[0] + a_ref[1] + h_ref[0]
        x3 = jnp.maximum(tot * dinv + b_ref[...], 0.0)
        bsum = jnp.sum(x3, axis=0, keepdims=True)
        bmax = jnp.max(x3, axis=0, keepdims=True)

        @pl.when(i == 0)
        def _init():
            sum_ref[...] = bsum
            max_ref[...] = bmax

        @pl.when(i > 0)
        def _accum():
            sum_ref[...] += bsum
            max_ref[...] = jnp.maximum(max_ref[...], bmax)

        @pl.when(i == N // _BM - 1)
        def _final():
            mean = sum_ref[...] * (1.0 / N)
            o_ref[...] = (jnp.dot(mean, wm_ref[...],
                                  preferred_element_type=jnp.float32)
                          + jnp.dot(max_ref[...], wx_ref[...],
                                    preferred_element_type=jnp.float32)
                          + bf_ref[...])
    return pl.pallas_call(
        body,
        grid=(N // _BM,),
        in_specs=[
            pl.BlockSpec((CORES, _BM, D), lambda i: (0, i, 0)),
            pl.BlockSpec((CORES, _BM, D), lambda i: (0, i, 0)),
            pl.BlockSpec((CORES, _BM, 1), lambda i: (0, i, 0)),
            pl.BlockSpec((1, D), lambda i: (0, 0)),
            pl.BlockSpec((D, D), lambda i: (0, 0)),
            pl.BlockSpec((D, D), lambda i: (0, 0)),
            pl.BlockSpec((1, D), lambda i: (0, 0)),
        ],
        out_specs=pl.BlockSpec((1, D), lambda i: (0, 0)),
        out_shape=jax.ShapeDtypeStruct((1, D), jnp.float32),
        scratch_shapes=[
            pltpu.VMEM((1, D), jnp.float32),
            pltpu.VMEM((1, D), jnp.float32),
        ],
    )(acc, hp, deg, b, Wfm, Wfx, bfc)


def kernel(node_features, edge_index, W1, b1, W2, b2, W_fc, b_fc):
    ei = edge_index.astype(jnp.int32)
    npad = E_PAD - E
    src = jnp.concatenate([ei[0], jnp.zeros((npad,), jnp.int32)])
    dst = jnp.concatenate([ei[1], jnp.full((npad,), PAD_DST, jnp.int32)])
    dst2d = dst.reshape(E_PAD // CHUNK, CHUNK)
    dst3d = dst.reshape(NT, 1, CHUNK)

    zeros = jnp.zeros((N_ACC, D), jnp.float32)
    src2 = jnp.stack([src, src + N])

    deg = _sc_degree(dst2d)[:, :N].reshape(CORES, N, 1)
    hp1 = _tc_first(node_features, W1, deg)
    acc1 = _sc_message(hp1.reshape(CORES * N, D), src2, dst3d, zeros)
    hp2 = _tc_mid(acc1, hp1, deg, b1.reshape(1, D), W2)
    acc2 = _sc_message(hp2.reshape(CORES * N, D), src2, dst3d, zeros)
    return _tc_last(acc2, hp2, deg, b2.reshape(1, D),
                    W_fc[:D], W_fc[D:], b_fc.reshape(1, D))


# speedup vs baseline: 1.6027x; 1.0222x over previous
import functools

import jax
import jax.numpy as jnp
from jax import lax
from jax.experimental import pallas as pl
from jax.experimental.pallas import tpu as pltpu
from jax.experimental.pallas import tpu_sc as plsc

N = 10000
E = 320000
D = 128
TILES = 16
CORES = 2
CHUNK = 128
NCHUNK = 80
NC0 = 152
NC1 = 8
NT = (NC0 + NC1) * TILES
EPT = NCHUNK * CHUNK
E_PAD = EPT * TILES * CORES
N_ACC = 10112
N_DEG = 10240
PAD_DST = 10008
ROWS_T = 624
ROWS_LAST = N - 15 * ROWS_T
ACC_T = N_ACC // TILES
DEG_T = N_DEG // TILES

_MESH = plsc.VectorSubcoreMesh(core_axis_name="c", subcore_axis_name="s")


@functools.partial(
    pl.kernel,
    out_type=jax.ShapeDtypeStruct((CORES, N_DEG), jnp.float32),
    mesh=_MESH,
    scratch_types=[
        pltpu.VMEM((NCHUNK, CHUNK), jnp.int32),
        pltpu.VMEM((CHUNK,), jnp.float32),
        pltpu.VMEM((DEG_T,), jnp.float32),
        pltpu.VMEM_SHARED((N_DEG,), jnp.float32),
    ],
)
def _sc_degree(dst_hbm, deg_hbm, dst_v, ones_v, zer_v, deg_sh):
    c = lax.axis_index("c")
    s = lax.axis_index("s")
    for i in range(CHUNK // 16):
        ones_v[pl.ds(i * 16, 16)] = jnp.ones((16,), jnp.float32)
    def _z(i, _):
        zer_v[pl.ds(i * 16, 16)] = jnp.zeros((16,), jnp.float32)
        return 0
    lax.fori_loop(0, DEG_T // 16, _z, 0)
    pltpu.sync_copy(zer_v, deg_sh.at[pl.ds(s * DEG_T, DEG_T)])
    pltpu.sync_copy(
        dst_hbm.at[pl.ds((c * TILES + s) * NCHUNK, NCHUNK), :], dst_v)
    plsc.subcore_barrier()

    def _body(j, _):
        pltpu.sync_copy(ones_v, deg_sh.at[dst_v.at[j]], add=True)
        return 0
    lax.fori_loop(0, NCHUNK, _body, 0)
    plsc.subcore_barrier()
    pltpu.sync_copy(deg_sh.at[pl.ds(s * DEG_T, DEG_T)],
                    deg_hbm.at[c, pl.ds(s * DEG_T, DEG_T)])


@functools.partial(
    pl.kernel,
    out_type=jax.ShapeDtypeStruct((CORES, N, D), jnp.float32),
    mesh=_MESH,
    scratch_types=[
        pltpu.VMEM((CHUNK,), jnp.int32),
        pltpu.VMEM((CHUNK,), jnp.int32),
        pltpu.VMEM((CHUNK,), jnp.int32),
        pltpu.VMEM((1, CHUNK), jnp.int32),
        pltpu.VMEM((1, CHUNK), jnp.int32),
        pltpu.VMEM((1, CHUNK), jnp.int32),
        pltpu.VMEM((CHUNK, D), jnp.float32),
        pltpu.VMEM((CHUNK, D), jnp.float32),
        pltpu.VMEM((CHUNK, D), jnp.float32),
        pltpu.VMEM_SHARED((N_ACC, D), jnp.float32),
        pltpu.SemaphoreType.DMA,
        pltpu.SemaphoreType.DMA,
        pltpu.SemaphoreType.DMA,
    ],
)
def _sc_message(hp_hbm, src_hbm, dst_hbm, zeros_hbm, out_hbm,
                sidx0_v, sidx1_v, sidx2_v, didx0_v, didx1_v, didx2_v,
                rows0_v, rows1_v, rows2_v, acc_sh, sem_g, sem_i, sem_s):
    c = lax.axis_index("c")
    s = lax.axis_index("s")
    pltpu.sync_copy(zeros_hbm.at[pl.ds(s * ACC_T, ACC_T), :],
                    acc_sh.at[pl.ds(s * ACC_T, ACC_T), :])
    plsc.subcore_barrier()

    def _fetch_idx(t, ibuf, dbuf):
        pltpu.async_copy(src_hbm.at[c, pl.ds(t * CHUNK, CHUNK)], ibuf, sem_i)
        pltpu.async_copy(dst_hbm.at[t], dbuf, sem_i)

    def _iwait(ibuf, dbuf):
        pltpu.make_async_copy(src_hbm.at[0, pl.ds(0, CHUNK)],
                              ibuf, sem_i).wait()
        pltpu.make_async_copy(dst_hbm.at[0], dbuf, sem_i).wait()

    def _gather(ibuf, buf):
        pltpu.async_copy(hp_hbm.at[ibuf], buf, sem_g)

    def _gwait(ibuf, buf):
        pltpu.make_async_copy(hp_hbm.at[ibuf], buf, sem_g).wait()

    def _scatter(buf, dbuf):
        pltpu.async_copy(buf, acc_sh.at[dbuf.at[0]], sem_s, add=True)

    def _swait():
        pltpu.make_async_copy(rows0_v, acc_sh.at[didx0_v.at[0]],
                              sem_s).wait()

    bufs = [(sidx0_v, didx0_v, rows0_v),
            (sidx1_v, didx1_v, rows1_v),
            (sidx2_v, didx2_v, rows2_v)]

    def _run(nck, t0):
        _fetch_idx(t0, bufs[0][0], bufs[0][1])
        _iwait(bufs[0][0], bufs[0][1])
        _gather(bufs[0][0], bufs[0][2])
        _fetch_idx(t0 + 1, bufs[1][0], bufs[1][1])

        def _body(g, _):
            def _step(cur, nxt, nxt2):
                _gwait(cur[0], cur[2])

                @pl.when(g + 1 < nck)
                def _launch_next():
                    _iwait(nxt[0], nxt[1])

                    @pl.when(g >= 2)
                    def _w():
                        _swait()
                    _gather(nxt[0], nxt[2])

                @pl.when(g + 2 < nck)
                def _prefetch_idx():
                    _fetch_idx(t0 + g + 2, nxt2[0], nxt2[1])
                _scatter(cur[2], cur[1])

            @pl.when(g % 3 == 0)
            def _r0():
                _step(bufs[0], bufs[1], bufs[2])

            @pl.when(g % 3 == 1)
            def _r1():
                _step(bufs[1], bufs[2], bufs[0])

            @pl.when(g % 3 == 2)
            def _r2():
                _step(bufs[2], bufs[0], bufs[1])
            return 0
        lax.fori_loop(0, nck, _body, 0)
        _swait()
        _swait()
        _swait()

    @pl.when(c == 0)
    def _core0():
        _run(NC0, s * NC0)

    @pl.when(c == 1)
    def _core1():
        _run(NC1, TILES * NC0 + s * NC1)
    plsc.subcore_barrier()

    @pl.when(s < 15)
    def _drain():
        pltpu.sync_copy(acc_sh.at[pl.ds(s * ROWS_T, ROWS_T), :],
                        out_hbm.at[c, pl.ds(s * ROWS_T, ROWS_T), :])

    @pl.when(s == 15)
    def _drain_last():
        pltpu.sync_copy(acc_sh.at[pl.ds(15 * ROWS_T, ROWS_LAST), :],
                        out_hbm.at[c, pl.ds(15 * ROWS_T, ROWS_LAST), :])


_BM = 1000


def _tc_first(x, W1, deg):
    def body(x_ref, w_ref, d_ref, o_ref):
        dinv = lax.rsqrt(d_ref[0] + d_ref[1] + 1.0)
        h = jnp.dot(x_ref[...], w_ref[...],
                    preferred_element_type=jnp.float32) * dinv
        o_ref[0] = h
        o_ref[1] = h
    return pl.pallas_call(
        body,
        grid=(N // _BM,),
        in_specs=[
            pl.BlockSpec((_BM, D), lambda i: (i, 0)),
            pl.BlockSpec((D, D), lambda i: (0, 0)),
            pl.BlockSpec((CORES, _BM, 1), lambda i: (0, i, 0)),
        ],
        out_specs=pl.BlockSpec((CORES, _BM, D), lambda i: (0, i, 0)),
        out_shape=jax.ShapeDtypeStruct((CORES, N, D), jnp.float32),
    )(x, W1, deg)


def _tc_mid(acc, hp, deg, b, W2):
    def body(a_ref, h_ref, d_ref, b_ref, w_ref, o_ref):
        dinv = lax.rsqrt(d_ref[0] + d_ref[1] + 1.0)
        tot = a_ref[0] + a_ref[1] + h_ref[0]
        x2 = jnp.maximum(tot * dinv + b_ref[...], 0.0)
        h = jnp.dot(x2, w_ref[...],
                    preferred_element_type=jnp.float32) * dinv
        o_ref[0] = h
        o_ref[1] = h
    return pl.pallas_call(
        body,
        grid=(N // _BM,),
        in_specs=[
            pl.BlockSpec((CORES, _BM, D), lambda i: (0, i, 0)),
            pl.BlockSpec((CORES, _BM, D), lambda i: (0, i, 0)),
            pl.BlockSpec((CORES, _BM, 1), lambda i: (0, i, 0)),
            pl.BlockSpec((1, D), lambda i: (0, 0)),
            pl.BlockSpec((D, D), lambda i: (0, 0)),
        ],
        out_specs=pl.BlockSpec((CORES, _BM, D), lambda i: (0, i, 0)),
        out_shape=jax.ShapeDtypeStruct((CORES, N, D), jnp.float32),
    )(acc, hp, deg, b, W2)


def _tc_last(acc, hp, deg, b, Wfm, Wfx, bfc):
    def body(a_ref, h_ref, d_ref, b_ref, wm_ref, wx_ref, bf_ref, o_ref,
             sum_ref, max_ref):
        i = pl.program_id(0)
        dinv = lax.rsqrt(d_ref[0] + d_ref[1] + 1.0)
        tot = a_ref[0] + a_ref[1] + h_ref[0]
        x3 = jnp.maximum(tot * dinv + b_ref[...], 0.0)
        bsum = jnp.sum(x3, axis=0, keepdims=True)
        bmax = jnp.max(x3, axis=0, keepdims=True)

        @pl.when(i == 0)
        def _init():
            sum_ref[...] = bsum
            max_ref[...] = bmax

        @pl.when(i > 0)
        def _accum():
            sum_ref[...] += bsum
            max_ref[...] = jnp.maximum(max_ref[...], bmax)

        @pl.when(i == N // _BM - 1)
        def _final():
            mean = sum_ref[...] * (1.0 / N)
            o_ref[...] = (jnp.dot(mean, wm_ref[...],
                                  preferred_element_type=jnp.float32)
                          + jnp.dot(max_ref[...], wx_ref[...],
                                    preferred_element_type=jnp.float32)
                          + bf_ref[...])
    return pl.pallas_call(
        body,
        grid=(N // _BM,),
        in_specs=[
            pl.BlockSpec((CORES, _BM, D), lambda i: (0, i, 0)),
            pl.BlockSpec((CORES, _BM, D), lambda i: (0, i, 0)),
            pl.BlockSpec((CORES, _BM, 1), lambda i: (0, i, 0)),
            pl.BlockSpec((1, D), lambda i: (0, 0)),
            pl.BlockSpec((D, D), lambda i: (0, 0)),
            pl.BlockSpec((D, D), lambda i: (0, 0)),
            pl.BlockSpec((1, D), lambda i: (0, 0)),
        ],
        out_specs=pl.BlockSpec((1, D), lambda i: (0, 0)),
        out_shape=jax.ShapeDtypeStruct((1, D), jnp.float32),
        scratch_shapes=[
            pltpu.VMEM((1, D), jnp.float32),
            pltpu.VMEM((1, D), jnp.float32),
        ],
    )(acc, hp, deg, b, Wfm, Wfx, bfc)


def kernel(node_features, edge_index, W1, b1, W2, b2, W_fc, b_fc):
    ei = edge_index.astype(jnp.int32)
    npad = E_PAD - E
    src = jnp.concatenate([ei[0], jnp.zeros((npad,), jnp.int32)])
    dst = jnp.concatenate([ei[1], jnp.full((npad,), PAD_DST, jnp.int32)])
    dst2d = dst.reshape(E_PAD // CHUNK, CHUNK)
    dst3d = dst.reshape(NT, 1, CHUNK)

    zeros = jnp.zeros((N_ACC, D), jnp.float32)
    src2 = jnp.stack([src, src + N])

    deg = _sc_degree(dst2d)[:, :N].reshape(CORES, N, 1)
    hp1 = _tc_first(node_features, W1, deg)
    acc1 = _sc_message(hp1.reshape(CORES * N, D), src2, dst3d, zeros)
    hp2 = _tc_mid(acc1, hp1, deg, b1.reshape(1, D), W2)
    acc2 = _sc_message(hp2.reshape(CORES * N, D), src2, dst3d, zeros)
    return _tc_last(acc2, hp2, deg, b2.reshape(1, D),
                    W_fc[:D], W_fc[D:], b_fc.reshape(1, D))
